# baseline (speedup 1.0000x reference)
import dataclasses
import functools

import jax
import jax.numpy as jnp
from jax import lax
from jax.experimental import pallas as pl
from jax.experimental.pallas import tpu as pltpu
from jax.experimental.pallas import tpu_sc as plsc

_NTILES = 16
_NCORES = 2
_NW = _NCORES * _NTILES
_KD = 512
_KE = 2048
_CPT = 4


def _round_up(v, m):
    return (v + m - 1) // m * m


def _sc_params():
    cp = pltpu.CompilerParams()
    if "needs_layout_passes" in pltpu.CompilerParams.__dataclass_fields__:
        cp = dataclasses.replace(cp, needs_layout_passes=False)
    return cp


def _sc_degree(dst_all, n_pad, e_pad):
    tpt = e_pad // _NW
    nchunk = tpt // _KD
    mesh = plsc.VectorSubcoreMesh(core_axis_name="c", subcore_axis_name="s")

    @functools.partial(
        pl.kernel,
        out_type=jax.ShapeDtypeStruct((_NW * n_pad,), jnp.float32),
        mesh=mesh,
        compiler_params=_sc_params(),
        scratch_types=[
            pltpu.VMEM((_KD,), jnp.int32),
            pltpu.VMEM((n_pad,), jnp.float32),
        ],
    )
    def deg_kernel(dst_hbm, out_hbm, dst_v, hist_v):
        c = lax.axis_index("c")
        s = lax.axis_index("s")
        t = s * _NCORES + c

        @pl.loop(0, n_pad // 16)
        def _(i):
            hist_v[pl.ds(i * 16, 16)] = jnp.zeros((16,), jnp.float32)

        ones16 = jnp.ones((16,), jnp.float32)
        base = t * tpt

        @pl.loop(0, nchunk)
        def _(g):
            pltpu.sync_copy(dst_hbm.at[pl.ds(base + g * _KD, _KD)], dst_v)
            for j in range(_KD // 16):
                dst16 = dst_v[pl.ds(j * 16, 16)]
                plsc.addupdate_scatter(hist_v, [dst16], ones16)

        pltpu.sync_copy(hist_v, out_hbm.at[pl.ds(t * n_pad, n_pad)])

    return deg_kernel(dst_all)


def _sc_scatter(yt_flat, src_all, dst_all, n_pad, e_pad):
    nchunk = e_pad // _KE
    mesh = plsc.VectorSubcoreMesh(core_axis_name="c", subcore_axis_name="s")

    @functools.partial(
        pl.kernel,
        out_type=jax.ShapeDtypeStruct((_NW * _CPT * n_pad,), jnp.float32),
        mesh=mesh,
        compiler_params=_sc_params(),
        scratch_types=[
            pltpu.VMEM((2, _KE), jnp.int32),
            pltpu.VMEM((2, _KE), jnp.int32),
            pltpu.VMEM((_CPT * n_pad,), jnp.float32),
            pltpu.VMEM((_CPT * n_pad,), jnp.float32),
            pltpu.SemaphoreType.DMA,
            pltpu.SemaphoreType.DMA,
            pltpu.SemaphoreType.DMA,
            pltpu.SemaphoreType.DMA,
        ],
    )
    def edge_kernel(yt_hbm, src_hbm, dst_hbm, out_hbm,
                    src_v, dst_v, ycols_v, acc_v,
                    sem_s0, sem_s1, sem_d0, sem_d1):
        c = lax.axis_index("c")
        s = lax.axis_index("s")
        t = s * _NCORES + c
        sems = ((sem_s0, sem_d0), (sem_s1, sem_d1))

        def start(g, buf):
            ss, sd = sems[buf]
            pltpu.async_copy(src_hbm.at[pl.ds(g * _KE, _KE)],
                             src_v.at[buf], ss)
            pltpu.async_copy(dst_hbm.at[pl.ds(g * _KE, _KE)],
                             dst_v.at[buf], sd)

        def wait(g, buf):
            ss, sd = sems[buf]
            pltpu.make_async_copy(src_hbm.at[pl.ds(g * _KE, _KE)],
                                  src_v.at[buf], ss).wait()
            pltpu.make_async_copy(dst_hbm.at[pl.ds(g * _KE, _KE)],
                                  dst_v.at[buf], sd).wait()

        def process(buf):
            for j in range(0, _KE // 16, 4):
                sg = [src_v[buf, pl.ds((j + k) * 16, 16)] for k in range(4)]
                dg = [dst_v[buf, pl.ds((j + k) * 16, 16)] for k in range(4)]
                vg = [[plsc.load_gather(ycols_v, [sg[k] + f * n_pad])
                       for f in range(_CPT)] for k in range(4)]
                for k in range(4):
                    for f in range(_CPT):
                        plsc.addupdate_scatter(acc_v, [dg[k] + f * n_pad],
                                               vg[k][f])

        start(0, 0)
        start(1, 1)

        pltpu.sync_copy(yt_hbm.at[pl.ds(t * _CPT * n_pad, _CPT * n_pad)],
                        ycols_v)

        @pl.loop(0, (_CPT * n_pad) // 16)
        def _(i):
            acc_v[pl.ds(i * 16, 16)] = jnp.zeros((16,), jnp.float32)

        @pl.loop(0, nchunk, step=2)
        def _(g):
            for buf in range(2):
                wait(g + buf, buf)
                process(buf)

                @pl.when(g + 2 + buf < nchunk)
                def _():
                    start(g + 2 + buf, buf)

        pltpu.sync_copy(acc_v, out_hbm.at[pl.ds(t * _CPT * n_pad,
                                                _CPT * n_pad)])

    return edge_kernel(yt_flat, src_all, dst_all)


def _tc_prep(deg2d, xt_pad, n):
    d, n_pad = xt_pad.shape

    def body(deg_ref, xt_ref, yt_ref, dinv_ref):
        deg = jnp.sum(deg_ref[...], axis=0, keepdims=True)
        col = lax.broadcasted_iota(jnp.int32, (1, n_pad), 1)
        dinv = jnp.where(col < n, lax.rsqrt(deg), 0.0)
        dinv_ref[...] = dinv
        yt_ref[...] = xt_ref[...] * dinv

    return pl.pallas_call(
        body,
        out_shape=[jax.ShapeDtypeStruct((d, n_pad), jnp.float32),
                   jax.ShapeDtypeStruct((1, n_pad), jnp.float32)],
    )(deg2d, xt_pad)


def _tc_agg(acct, dinv, w, b, n, d, h, tile):
    n_pad = acct.shape[1]
    steps = n_pad // tile
    npad = n_pad - n

    def body(acc_ref, dinv_ref, w_ref, b_ref, agg_ref, stats_ref, scr):
        i = pl.program_id(0)
        aggxt = acc_ref[...] * dinv_ref[...]
        agg = lax.dot_general(aggxt, w_ref[...],
                              (((0,), (0,)), ((), ())),
                              preferred_element_type=jnp.float32) + b_ref[...]
        agg_ref[...] = agg

        @pl.when(i == 0)
        def _():
            scr[...] = jnp.zeros_like(scr)

        scr[0:1, :] = scr[0:1, :] + jnp.sum(agg, axis=0, keepdims=True)
        scr[1:2, :] = scr[1:2, :] + jnp.sum(agg * agg, axis=0, keepdims=True)
        bb = b_ref[...]
        stats_ref[0:1, :] = scr[0:1, :] - npad * bb
        stats_ref[1:2, :] = scr[1:2, :] - npad * bb * bb

    return pl.pallas_call(
        body,
        grid=(steps,),
        in_specs=[
            pl.BlockSpec((d, tile), lambda i: (0, i)),
            pl.BlockSpec((1, tile), lambda i: (0, i)),
            pl.BlockSpec((d, h), lambda i: (0, 0)),
            pl.BlockSpec((1, h), lambda i: (0, 0)),
        ],
        out_specs=[
            pl.BlockSpec((tile, h), lambda i: (i, 0)),
            pl.BlockSpec((2, h), lambda i: (0, 0)),
        ],
        out_shape=[jax.ShapeDtypeStruct((n_pad, h), jnp.float32),
                   jax.ShapeDtypeStruct((2, h), jnp.float32)],
        scratch_shapes=[pltpu.VMEM((2, h), jnp.float32)],
    )(acct, dinv, w, b)


def _tc_head(agg, stats, gamma, beta, lin_w, lin_b, n, h, o, tile):
    n_pad = agg.shape[0]
    steps = n_pad // tile
    inv_n = 1.0 / n

    def body(agg_ref, st_ref, g_ref, be_ref, lw_ref, lb_ref,
             prob_ref, emb_ref):
        mean = st_ref[0:1, :] * inv_n
        var = st_ref[1:2, :] * inv_n - mean * mean
        scale = g_ref[...] * lax.rsqrt(var + 1e-5)
        shift = be_ref[...] - mean * scale
        xn = agg_ref[...] * scale + shift
        z = jnp.dot(xn, lw_ref[...],
                    preferred_element_type=jnp.float32) + lb_ref[...]
        r = jnp.maximum(z, 0.0)
        m = jnp.max(r, axis=1, keepdims=True)
        e = jnp.exp(r - m)
        prob_ref[...] = e / jnp.sum(e, axis=1, keepdims=True)
        emb_ref[...] = r

    return pl.pallas_call(
        body,
        grid=(steps,),
        in_specs=[
            pl.BlockSpec((tile, h), lambda i: (i, 0)),
            pl.BlockSpec((2, h), lambda i: (0, 0)),
            pl.BlockSpec((1, h), lambda i: (0, 0)),
            pl.BlockSpec((1, h), lambda i: (0, 0)),
            pl.BlockSpec((h, o), lambda i: (0, 0)),
            pl.BlockSpec((1, o), lambda i: (0, 0)),
        ],
        out_specs=[
            pl.BlockSpec((tile, o), lambda i: (i, 0)),
            pl.BlockSpec((tile, o), lambda i: (i, 0)),
        ],
        out_shape=[jax.ShapeDtypeStruct((n_pad, o), jnp.float32),
                   jax.ShapeDtypeStruct((n_pad, o), jnp.float32)],
    )(agg, stats, gamma, beta, lin_w, lin_b)


def kernel(node_feature, edge_index, W, b, gamma, beta, lin_W, lin_b):
    n, d = node_feature.shape
    e = edge_index.shape[1]
    h = W.shape[1]
    o = lin_W.shape[1]

    n_pad = _round_up(n + 1, 1024)
    e_all = e + n
    e_pad = _round_up(e_all, _NW * _KD)

    sl = jnp.arange(n, dtype=jnp.int32)
    npad_e = e_pad - e_all
    nbins = n_pad - n
    pad_src = jnp.zeros((npad_e,), jnp.int32)
    pad_dst = n + (jnp.arange(npad_e, dtype=jnp.int32) % nbins)
    src_all = jnp.concatenate([edge_index[0], sl, pad_src])
    dst_all = jnp.concatenate([edge_index[1], sl, pad_dst])

    deg_flat = _sc_degree(dst_all, n_pad, e_pad)

    xt_pad = jnp.zeros((d, n_pad), node_feature.dtype)
    xt_pad = lax.dynamic_update_slice(xt_pad, node_feature.T, (0, 0))
    yt, dinv = _tc_prep(deg_flat.reshape(_NW, n_pad), xt_pad, n)

    acct_flat = _sc_scatter(yt.reshape(-1), src_all, dst_all, n_pad, e_pad)
    acct = acct_flat.reshape(d, n_pad)

    tile = 1024
    agg, stats = _tc_agg(acct, dinv, W, b.reshape(1, h), n, d, h, tile)
    prob, emb = _tc_head(agg, stats, gamma.reshape(1, h), beta.reshape(1, h),
                         lin_W, lin_b.reshape(1, o), n, h, o, tile)
    return (prob[:n], emb[:n])


# speedup vs baseline: 1.3670x; 1.3670x over previous
import dataclasses
import functools

import jax
import jax.numpy as jnp
from jax import lax
from jax.experimental import pallas as pl
from jax.experimental.pallas import tpu as pltpu
from jax.experimental.pallas import tpu_sc as plsc

_NTILES = 16
_NCORES = 2
_NW = _NCORES * _NTILES
_KD = 512
_KE = 1024
_CPT = 4


def _round_up(v, m):
    return (v + m - 1) // m * m


def _sc_params():
    cp = pltpu.CompilerParams()
    if "needs_layout_passes" in pltpu.CompilerParams.__dataclass_fields__:
        cp = dataclasses.replace(cp, needs_layout_passes=False)
    return cp


def _sc_degree(dst_all, n_pad, e_pad):
    tpt = e_pad // _NW
    nchunk = tpt // _KD
    mesh = plsc.VectorSubcoreMesh(core_axis_name="c", subcore_axis_name="s")

    @functools.partial(
        pl.kernel,
        out_type=jax.ShapeDtypeStruct((_NW * n_pad,), jnp.float32),
        mesh=mesh,
        compiler_params=_sc_params(),
        scratch_types=[
            pltpu.VMEM((_KD,), jnp.int32),
            pltpu.VMEM((n_pad,), jnp.float32),
        ],
    )
    def deg_kernel(dst_hbm, out_hbm, dst_v, hist_v):
        c = lax.axis_index("c")
        s = lax.axis_index("s")
        t = s * _NCORES + c

        @pl.loop(0, n_pad // 16)
        def _(i):
            hist_v[pl.ds(i * 16, 16)] = jnp.zeros((16,), jnp.float32)

        ones16 = jnp.ones((16,), jnp.float32)
        base = t * tpt

        @pl.loop(0, nchunk)
        def _(g):
            pltpu.sync_copy(dst_hbm.at[pl.ds(base + g * _KD, _KD)], dst_v)
            for j in range(_KD // 16):
                dst16 = dst_v[pl.ds(j * 16, 16)]
                plsc.addupdate_scatter(hist_v, [dst16], ones16)

        pltpu.sync_copy(hist_v, out_hbm.at[pl.ds(t * n_pad, n_pad)])

    return deg_kernel(dst_all)


def _sc_scatter(yt_flat, src_all, dst_all, n_pad, e_pad):
    nchunk = e_pad // _KE
    mesh = plsc.VectorSubcoreMesh(core_axis_name="c", subcore_axis_name="s")

    @functools.partial(
        pl.kernel,
        out_type=jax.ShapeDtypeStruct((_NW * _CPT * n_pad,), jnp.float32),
        mesh=mesh,
        compiler_params=_sc_params(),
        scratch_types=[
            pltpu.VMEM((2, _KE), jnp.int32),
            pltpu.VMEM((2, _KE), jnp.int32),
            pltpu.VMEM((_CPT * n_pad,), jnp.float32),
            pltpu.VMEM((_CPT * n_pad,), jnp.float32),
            pltpu.SemaphoreType.DMA,
            pltpu.SemaphoreType.DMA,
            pltpu.SemaphoreType.DMA,
            pltpu.SemaphoreType.DMA,
        ],
    )
    def edge_kernel(yt_hbm, src_hbm, dst_hbm, out_hbm,
                    src_v, dst_v, ycols_v, acc_v,
                    sem_s0, sem_s1, sem_d0, sem_d1):
        c = lax.axis_index("c")
        s = lax.axis_index("s")
        t = s * _NCORES + c
        sems = ((sem_s0, sem_d0), (sem_s1, sem_d1))

        def start(g, buf):
            ss, sd = sems[buf]
            pltpu.async_copy(src_hbm.at[pl.ds(g * _KE, _KE)],
                             src_v.at[buf], ss)
            pltpu.async_copy(dst_hbm.at[pl.ds(g * _KE, _KE)],
                             dst_v.at[buf], sd)

        def wait(g, buf):
            ss, sd = sems[buf]
            pltpu.make_async_copy(src_hbm.at[pl.ds(g * _KE, _KE)],
                                  src_v.at[buf], ss).wait()
            pltpu.make_async_copy(dst_hbm.at[pl.ds(g * _KE, _KE)],
                                  dst_v.at[buf], sd).wait()

        def process(buf):
            for j in range(0, _KE // 16, 4):
                sg = [src_v[buf, pl.ds((j + k) * 16, 16)] for k in range(4)]
                dg = [dst_v[buf, pl.ds((j + k) * 16, 16)] for k in range(4)]
                vg = [[plsc.load_gather(ycols_v, [sg[k] + f * n_pad])
                       for f in range(_CPT)] for k in range(4)]
                for k in range(4):
                    for f in range(_CPT):
                        plsc.addupdate_scatter(acc_v, [dg[k] + f * n_pad],
                                               vg[k][f])

        start(0, 0)
        start(1, 1)

        pltpu.sync_copy(yt_hbm.at[pl.ds(t * _CPT * n_pad, _CPT * n_pad)],
                        ycols_v)

        @pl.loop(0, (_CPT * n_pad) // 16)
        def _(i):
            acc_v[pl.ds(i * 16, 16)] = jnp.zeros((16,), jnp.float32)

        @pl.loop(0, nchunk, step=2)
        def _(g):
            for buf in range(2):
                wait(g + buf, buf)
                process(buf)

                @pl.when(g + 2 + buf < nchunk)
                def _():
                    start(g + 2 + buf, buf)

        pltpu.sync_copy(acc_v, out_hbm.at[pl.ds(t * _CPT * n_pad,
                                                _CPT * n_pad)])

    return edge_kernel(yt_flat, src_all, dst_all)


def _tc_prep(deg2d, xt_pad, n):
    d, n_pad = xt_pad.shape

    def body(deg_ref, xt_ref, yt_ref, dinv_ref):
        deg = jnp.sum(deg_ref[...], axis=0, keepdims=True)
        col = lax.broadcasted_iota(jnp.int32, (1, n_pad), 1)
        dinv = jnp.where(col < n, lax.rsqrt(deg), 0.0)
        dinv_ref[...] = dinv
        yt_ref[...] = xt_ref[...] * dinv

    return pl.pallas_call(
        body,
        out_shape=[jax.ShapeDtypeStruct((d, n_pad), jnp.float32),
                   jax.ShapeDtypeStruct((1, n_pad), jnp.float32)],
    )(deg2d, xt_pad)


def _tc_agg(acct, dinv, w, b, n, d, h, tile):
    n_pad = acct.shape[1]
    steps = n_pad // tile
    npad = n_pad - n

    def body(acc_ref, dinv_ref, w_ref, b_ref, agg_ref, stats_ref, scr):
        i = pl.program_id(0)
        aggxt = acc_ref[...] * dinv_ref[...]
        agg = lax.dot_general(aggxt, w_ref[...],
                              (((0,), (0,)), ((), ())),
                              preferred_element_type=jnp.float32) + b_ref[...]
        agg_ref[...] = agg

        @pl.when(i == 0)
        def _():
            scr[...] = jnp.zeros_like(scr)

        scr[0:1, :] = scr[0:1, :] + jnp.sum(agg, axis=0, keepdims=True)
        scr[1:2, :] = scr[1:2, :] + jnp.sum(agg * agg, axis=0, keepdims=True)
        bb = b_ref[...]
        stats_ref[0:1, :] = scr[0:1, :] - npad * bb
        stats_ref[1:2, :] = scr[1:2, :] - npad * bb * bb

    return pl.pallas_call(
        body,
        grid=(steps,),
        in_specs=[
            pl.BlockSpec((d, tile), lambda i: (0, i)),
            pl.BlockSpec((1, tile), lambda i: (0, i)),
            pl.BlockSpec((d, h), lambda i: (0, 0)),
            pl.BlockSpec((1, h), lambda i: (0, 0)),
        ],
        out_specs=[
            pl.BlockSpec((tile, h), lambda i: (i, 0)),
            pl.BlockSpec((2, h), lambda i: (0, 0)),
        ],
        out_shape=[jax.ShapeDtypeStruct((n_pad, h), jnp.float32),
                   jax.ShapeDtypeStruct((2, h), jnp.float32)],
        scratch_shapes=[pltpu.VMEM((2, h), jnp.float32)],
    )(acct, dinv, w, b)


def _tc_head(agg, stats, gamma, beta, lin_w, lin_b, n, h, o, tile):
    n_pad = agg.shape[0]
    steps = n_pad // tile
    inv_n = 1.0 / n

    def body(agg_ref, st_ref, g_ref, be_ref, lw_ref, lb_ref,
             prob_ref, emb_ref):
        mean = st_ref[0:1, :] * inv_n
        var = st_ref[1:2, :] * inv_n - mean * mean
        scale = g_ref[...] * lax.rsqrt(var + 1e-5)
        shift = be_ref[...] - mean * scale
        xn = agg_ref[...] * scale + shift
        z = jnp.dot(xn, lw_ref[...],
                    preferred_element_type=jnp.float32) + lb_ref[...]
        r = jnp.maximum(z, 0.0)
        m = jnp.max(r, axis=1, keepdims=True)
        e = jnp.exp(r - m)
        prob_ref[...] = e / jnp.sum(e, axis=1, keepdims=True)
        emb_ref[...] = r

    return pl.pallas_call(
        body,
        grid=(steps,),
        in_specs=[
            pl.BlockSpec((tile, h), lambda i: (i, 0)),
            pl.BlockSpec((2, h), lambda i: (0, 0)),
            pl.BlockSpec((1, h), lambda i: (0, 0)),
            pl.BlockSpec((1, h), lambda i: (0, 0)),
            pl.BlockSpec((h, o), lambda i: (0, 0)),
            pl.BlockSpec((1, o), lambda i: (0, 0)),
        ],
        out_specs=[
            pl.BlockSpec((tile, o), lambda i: (i, 0)),
            pl.BlockSpec((tile, o), lambda i: (i, 0)),
        ],
        out_shape=[jax.ShapeDtypeStruct((n_pad, o), jnp.float32),
                   jax.ShapeDtypeStruct((n_pad, o), jnp.float32)],
    )(agg, stats, gamma, beta, lin_w, lin_b)


def kernel(node_feature, edge_index, W, b, gamma, beta, lin_W, lin_b):
    n, d = node_feature.shape
    e = edge_index.shape[1]
    h = W.shape[1]
    o = lin_W.shape[1]

    n_pad = _round_up(n + 1, 1024)
    e_all = e + n
    e_pad = _round_up(e_all, _NW * _KD)

    sl = jnp.arange(n, dtype=jnp.int32)
    npad_e = e_pad - e_all
    nbins = n_pad - n
    pad_src = jnp.zeros((npad_e,), jnp.int32)
    pad_dst = n + (jnp.arange(npad_e, dtype=jnp.int32) % nbins)
    src_all = jnp.concatenate([edge_index[0], sl, pad_src])
    dst_all = jnp.concatenate([edge_index[1], sl, pad_dst])

    deg_flat = _sc_degree(dst_all, n_pad, e_pad)

    xt_pad = jnp.zeros((d, n_pad), node_feature.dtype)
    xt_pad = lax.dynamic_update_slice(xt_pad, node_feature.T, (0, 0))
    yt, dinv = _tc_prep(deg_flat.reshape(_NW, n_pad), xt_pad, n)

    acct_flat = _sc_scatter(yt.reshape(-1), src_all, dst_all, n_pad, e_pad)
    acct = acct_flat.reshape(d, n_pad)

    tile = 1024
    agg, stats = _tc_agg(acct, dinv, W, b.reshape(1, h), n, d, h, tile)
    prob, emb = _tc_head(agg, stats, gamma.reshape(1, h), beta.reshape(1, h),
                         lin_W, lin_b.reshape(1, o), n, h, o, tile)
    return (prob[:n], emb[:n])
